# Initial kernel scaffold; baseline (speedup 1.0000x reference)
import jax
import jax.numpy as jnp
from jax.experimental import pallas as pl


def kernel(x, edge_index, edge_weight, W1, b1, W2, b2):
    raise NotImplementedError("write your pallas kernel here")


# speedup vs baseline: 6.7186x; 6.7186x over previous
import dataclasses
import functools

import jax
import jax.numpy as jnp
from jax import lax
from jax.experimental import pallas as pl
from jax.experimental.pallas import tpu as pltpu
from jax.experimental.pallas import tpu_sc as plsc

NC = 2
NS = 16
L = 16
W = 128
NT = NC * NS

@functools.lru_cache(maxsize=None)
def _sc_mesh():
    return plsc.VectorSubcoreMesh(core_axis_name="c", subcore_axis_name="s",
                                  num_cores=NC, num_subcores=NS)

_SC_PARAMS = pltpu.CompilerParams()
if "needs_layout_passes" in pltpu.CompilerParams.__dataclass_fields__:
    _SC_PARAMS = dataclasses.replace(_SC_PARAMS, needs_layout_passes=False)


def _pad_up(v, m):
    return (v + m - 1) // m * m


def _sc_deg_body(dst_hbm, ew_hbm, deg_hbm, didx, ewb, stage, acc):
    c = lax.axis_index("c")
    s = lax.axis_index("s")
    ep = dst_hbm.shape[0]
    wpt = ep // (NT * W)
    np_ = acc.shape[0]
    rpt = np_ // NS

    @pl.loop(0, W)
    def _(e):
        for g in range(128 // L):
            stage[e, pl.ds(g * L, L)] = jnp.zeros((L,), jnp.float32)

    @pl.loop(0, rpt, step=W)
    def _(j):
        pltpu.sync_copy(stage, acc.at[pl.ds(s * rpt + j, W)])

    plsc.subcore_barrier()

    wid = s * NC + c

    @pl.loop(0, wpt)
    def _(w):
        base = (wid * wpt + w) * W
        pltpu.sync_copy(dst_hbm.at[pl.ds(base, W)], didx)
        pltpu.sync_copy(ew_hbm.at[pl.ds(base, W)], ewb)

        @pl.loop(0, W)
        def _(e):
            ev = jnp.full((L,), e, jnp.int32)
            wv = plsc.load_gather(ewb, [ev])
            for g in range(128 // L):
                stage[e, pl.ds(g * L, L)] = wv

        pltpu.sync_copy(stage, acc.at[didx], add=True)

    plsc.subcore_barrier()
    pltpu.sync_copy(acc.at[pl.ds(s * rpt, rpt)], deg_hbm.at[c, pl.ds(s * rpt, rpt)])


def _sc_deg(dst_pad, ew_pad, np_):
    out = jax.ShapeDtypeStruct((NC, np_, 128), jnp.float32)
    k = pl.kernel(
        _sc_deg_body,
        out_type=out,
        mesh=_sc_mesh(),
        scratch_types=[
            pltpu.VMEM((W,), jnp.int32),
            pltpu.VMEM((W,), jnp.float32),
            pltpu.VMEM((W, 128), jnp.float32),
            pltpu.VMEM_SHARED((np_, 128), jnp.float32),
        ],
        compiler_params=_SC_PARAMS,
    )
    return k(dst_pad, ew_pad)


def _sc_msg_body(hp_hbm, src_hbm, dst_hbm, ew_hbm, out_hbm,
                 sidx, didx, ewb, rows, acc):
    c = lax.axis_index("c")
    s = lax.axis_index("s")
    ep = src_hbm.shape[0]
    wpt = ep // (NT * W)
    np_ = acc.shape[0]
    rpt = np_ // NS

    @pl.loop(0, W)
    def _(e):
        for g in range(128 // L):
            rows[e, pl.ds(g * L, L)] = jnp.zeros((L,), jnp.float32)

    @pl.loop(0, rpt, step=W)
    def _(j):
        pltpu.sync_copy(rows, acc.at[pl.ds(s * rpt + j, W)])

    plsc.subcore_barrier()

    wid = s * NC + c

    @pl.loop(0, wpt)
    def _(w):
        base = (wid * wpt + w) * W
        pltpu.sync_copy(src_hbm.at[pl.ds(base, W)], sidx)
        pltpu.sync_copy(dst_hbm.at[pl.ds(base, W)], didx)
        pltpu.sync_copy(ew_hbm.at[pl.ds(base, W)], ewb)
        pltpu.sync_copy(hp_hbm.at[sidx], rows)

        @pl.loop(0, W)
        def _(e):
            ev = jnp.full((L,), e, jnp.int32)
            wv = plsc.load_gather(ewb, [ev])
            for g in range(128 // L):
                sl = (e, pl.ds(g * L, L))
                rows[sl] = rows[sl] * wv

        pltpu.sync_copy(rows, acc.at[didx], add=True)

    plsc.subcore_barrier()
    pltpu.sync_copy(acc.at[pl.ds(s * rpt, rpt)], out_hbm.at[c, pl.ds(s * rpt, rpt)])


def _sc_msg(hp, src_pad, dst_pad, ew_pad, np_):
    out = jax.ShapeDtypeStruct((NC, np_, 128), jnp.float32)
    k = pl.kernel(
        _sc_msg_body,
        out_type=out,
        mesh=_sc_mesh(),
        scratch_types=[
            pltpu.VMEM((W,), jnp.int32),
            pltpu.VMEM((W,), jnp.int32),
            pltpu.VMEM((W,), jnp.float32),
            pltpu.VMEM((W, 128), jnp.float32),
            pltpu.VMEM_SHARED((np_, 128), jnp.float32),
        ],
        compiler_params=_SC_PARAMS,
    )
    return k(hp, src_pad, dst_pad, ew_pad)


def _mm_body(x_ref, w_ref, o_ref):
    o_ref[...] = jnp.dot(x_ref[...], w_ref[...],
                         preferred_element_type=jnp.float32)


def _tc_mm(x, w):
    n, d = x.shape
    b = 512
    return pl.pallas_call(
        _mm_body,
        grid=(pl.cdiv(n, b),),
        in_specs=[pl.BlockSpec((b, d), lambda i: (i, 0)),
                  pl.BlockSpec((d, w.shape[1]), lambda i: (0, 0))],
        out_specs=pl.BlockSpec((b, w.shape[1]), lambda i: (i, 0)),
        out_shape=jax.ShapeDtypeStruct((n, w.shape[1]), jnp.float32),
    )(x, w)


def _dis_from_deg(deg_ref):
    d = deg_ref[0, :, 0] + deg_ref[1, :, 0]
    return lax.rsqrt(1.0 + d)


def _scale_body(h_ref, deg_ref, hp_ref):
    dis = _dis_from_deg(deg_ref)
    hp_ref[...] = dis[:, None] * h_ref[...]


def _tc_scale(h, deg2):
    n, d = h.shape
    b = 512
    return pl.pallas_call(
        _scale_body,
        grid=(pl.cdiv(n, b),),
        in_specs=[pl.BlockSpec((b, d), lambda i: (i, 0)),
                  pl.BlockSpec((NC, b, 128), lambda i: (0, i, 0))],
        out_specs=pl.BlockSpec((b, d), lambda i: (i, 0)),
        out_shape=jax.ShapeDtypeStruct((n, d), jnp.float32),
    )(h, deg2)


def _mid_body(acc_ref, h_ref, deg_ref, b_ref, w_ref, h2_ref, hp2_ref):
    dis = _dis_from_deg(deg_ref)[:, None]
    s = acc_ref[0] + acc_ref[1]
    out1 = dis * s + (dis * dis) * h_ref[...] + b_ref[...]
    g = jnp.maximum(out1, 0.0)
    h2 = jnp.dot(g, w_ref[...], preferred_element_type=jnp.float32)
    h2_ref[...] = h2
    hp2_ref[...] = dis * h2


def _tc_mid(acc2, h1, deg2, b1, w2):
    n, d = h1.shape
    b = 512
    return pl.pallas_call(
        _mid_body,
        grid=(pl.cdiv(n, b),),
        in_specs=[pl.BlockSpec((NC, b, d), lambda i: (0, i, 0)),
                  pl.BlockSpec((b, d), lambda i: (i, 0)),
                  pl.BlockSpec((NC, b, 128), lambda i: (0, i, 0)),
                  pl.BlockSpec((1, d), lambda i: (0, 0)),
                  pl.BlockSpec((d, d), lambda i: (0, 0))],
        out_specs=[pl.BlockSpec((b, d), lambda i: (i, 0)),
                   pl.BlockSpec((b, d), lambda i: (i, 0))],
        out_shape=[jax.ShapeDtypeStruct((n, d), jnp.float32),
                   jax.ShapeDtypeStruct((n, d), jnp.float32)],
    )(acc2, h1, deg2, b1, w2)


def _final_body(acc_ref, h_ref, deg_ref, b_ref, o_ref):
    dis = _dis_from_deg(deg_ref)[:, None]
    s = acc_ref[0] + acc_ref[1]
    o_ref[...] = dis * s + (dis * dis) * h_ref[...] + b_ref[...]


def _tc_final(acc2, h2, deg2, b2):
    n, d = h2.shape
    b = 512
    return pl.pallas_call(
        _final_body,
        grid=(pl.cdiv(n, b),),
        in_specs=[pl.BlockSpec((NC, b, d), lambda i: (0, i, 0)),
                  pl.BlockSpec((b, d), lambda i: (i, 0)),
                  pl.BlockSpec((NC, b, 128), lambda i: (0, i, 0)),
                  pl.BlockSpec((1, d), lambda i: (0, 0))],
        out_specs=pl.BlockSpec((b, d), lambda i: (i, 0)),
        out_shape=jax.ShapeDtypeStruct((n, d), jnp.float32),
    )(acc2, h2, deg2, b2)


def kernel(x, edge_index, edge_weight, W1, b1, W2, b2):
    n = x.shape[0]
    e = edge_weight.shape[0]
    ep = _pad_up(e, NT * W)
    np_ = _pad_up(n, NS * W)
    pad = ep - e

    src = jnp.concatenate([edge_index[0], jnp.zeros((pad,), jnp.int32)])
    dst = jnp.concatenate([edge_index[1], jnp.zeros((pad,), jnp.int32)])
    ew = jnp.concatenate([edge_weight, jnp.zeros((pad,), jnp.float32)])
    b1r = b1.reshape(1, -1)
    b2r = b2.reshape(1, -1)

    deg2 = _sc_deg(dst, ew, np_)
    h1 = _tc_mm(x, W1)
    hp1 = _tc_scale(h1, deg2)
    acc1 = _sc_msg(hp1, src, dst, ew, np_)[:, :n]
    h2, hp2 = _tc_mid(acc1, h1, deg2, b1r, W2)
    acc2 = _sc_msg(hp2, src, dst, ew, np_)[:, :n]
    return _tc_final(acc2, h2, deg2, b2r)


# speedup vs baseline: 7.5410x; 1.1224x over previous
import dataclasses
import functools

import jax
import jax.numpy as jnp
from jax import lax
from jax.experimental import pallas as pl
from jax.experimental.pallas import tpu as pltpu
from jax.experimental.pallas import tpu_sc as plsc

NC = 2
NS = 16
L = 16
W = 128
NT = NC * NS


@functools.lru_cache(maxsize=None)
def _sc_mesh():
    return plsc.VectorSubcoreMesh(core_axis_name="c", subcore_axis_name="s",
                                  num_cores=NC, num_subcores=NS)


_SC_PARAMS = pltpu.CompilerParams()
if "needs_layout_passes" in pltpu.CompilerParams.__dataclass_fields__:
    _SC_PARAMS = dataclasses.replace(_SC_PARAMS, needs_layout_passes=False)


def _pad_up(v, m):
    return (v + m - 1) // m * m


def _zero_rows(buf):
    @pl.loop(0, W)
    def _(e):
        for g in range(128 // L):
            buf[e, pl.ds(g * L, L)] = jnp.zeros((L,), jnp.float32)


def _zero_acc(buf, acc, s):
    rpt = acc.shape[0] // NS

    @pl.loop(0, rpt, step=W)
    def _(j):
        pltpu.sync_copy(buf, acc.at[pl.ds(s * rpt + j, W)])


def _sc_deg_body(dst_hbm, ew_hbm, deg_hbm, didx, ewb, stage, acc):
    c = lax.axis_index("c")
    s = lax.axis_index("s")
    ep = dst_hbm.shape[0]
    wpt = ep // (NT * W)
    np_ = acc.shape[0]
    rpt = np_ // NS
    wid = s * NC + c

    _zero_rows(stage)
    _zero_acc(stage, acc, s)
    plsc.subcore_barrier()

    @pl.loop(0, wpt)
    def _(w):
        base = (wid * wpt + w) * W
        pltpu.sync_copy(dst_hbm.at[pl.ds(base, W)], didx)
        pltpu.sync_copy(ew_hbm.at[pl.ds(base, W)], ewb)

        @pl.loop(0, W, step=4)
        def _(e):
            for k in range(4):
                ev = jnp.full((L,), e + k, jnp.int32)
                wv = plsc.load_gather(ewb, [ev])
                for g in range(128 // L):
                    stage[e + k, pl.ds(g * L, L)] = wv

        pltpu.sync_copy(stage, acc.at[didx], add=True)

    plsc.subcore_barrier()
    pltpu.sync_copy(acc.at[pl.ds(s * rpt, rpt)], deg_hbm.at[c, pl.ds(s * rpt, rpt)])


def _sc_deg(dst, ew, np_):
    out = jax.ShapeDtypeStruct((NC, np_, 128), jnp.float32)
    k = pl.kernel(
        _sc_deg_body,
        out_type=out,
        mesh=_sc_mesh(),
        scratch_types=[
            pltpu.VMEM((W,), jnp.int32),
            pltpu.VMEM((W,), jnp.float32),
            pltpu.VMEM((W, 128), jnp.float32),
            pltpu.VMEM_SHARED((np_, 128), jnp.float32),
        ],
        compiler_params=_SC_PARAMS,
    )
    return k(dst, ew)


def _sc_msg_body(hp_hbm, src_hbm, dst_hbm, ew_hbm, out_hbm,
                 sidx0, sidx1, didx0, didx1, ewb, rows, acc, semg0, semg1):
    c = lax.axis_index("c")
    s = lax.axis_index("s")
    ep = src_hbm.shape[0]
    wpt = ep // (NT * W)
    np_ = acc.shape[0]
    rpt = np_ // NS
    wid = s * NC + c

    _zero_rows(rows.at[0])
    _zero_acc(rows.at[0], acc, s)
    plsc.subcore_barrier()

    sidx = (sidx0, sidx1)
    didx = (didx0, didx1)
    semg = (semg0, semg1)

    def idx_load(w, b):
        base = (wid * wpt + w) * W
        pltpu.sync_copy(src_hbm.at[pl.ds(base, W)], sidx[b])
        pltpu.sync_copy(dst_hbm.at[pl.ds(base, W)], didx[b])
        pltpu.sync_copy(ew_hbm.at[pl.ds(base, W)], ewb.at[pl.ds(b * W, W)])

    def gather(b):
        pltpu.sync_copy(hp_hbm.at[sidx[b]], rows.at[b])

    def gather_wait(b):
        pass

    def scale(b):
        @pl.loop(0, W, step=4)
        def _(e):
            for k in range(4):
                ev = jnp.full((L,), b * W + e + k, jnp.int32)
                wv = plsc.load_gather(ewb, [ev])
                for g in range(128 // L):
                    sl = (b, e + k, pl.ds(g * L, L))
                    rows[sl] = rows[sl] * wv

    def consume(b):
        gather_wait(b)
        scale(b)
        pltpu.sync_copy(rows.at[b], acc.at[didx[b]], add=True)

    @pl.loop(0, wpt)
    def _(w):
        idx_load(w, 0)
        gather(0)
        consume(0)

    plsc.subcore_barrier()
    pltpu.sync_copy(acc.at[pl.ds(s * rpt, rpt)], out_hbm.at[c, pl.ds(s * rpt, rpt)])


def _sc_msg(hp, src, dst, ew, np_):
    out = jax.ShapeDtypeStruct((NC, np_, 128), jnp.float32)
    k = pl.kernel(
        _sc_msg_body,
        out_type=out,
        mesh=_sc_mesh(),
        scratch_types=[
            pltpu.VMEM((W,), jnp.int32),
            pltpu.VMEM((W,), jnp.int32),
            pltpu.VMEM((W,), jnp.int32),
            pltpu.VMEM((W,), jnp.int32),
            pltpu.VMEM((2 * W,), jnp.float32),
            pltpu.VMEM((2, W, 128), jnp.float32),
            pltpu.VMEM_SHARED((np_, 128), jnp.float32),
            pltpu.SemaphoreType.DMA,
            pltpu.SemaphoreType.DMA,
        ],
        compiler_params=_SC_PARAMS,
    )
    return k(hp, src, dst, ew)


def _dis_from_deg(deg_ref):
    d = deg_ref[0, :, 0] + deg_ref[1, :, 0]
    return lax.rsqrt(1.0 + d)


def _mm1_body(x_ref, w_ref, deg_ref, h_ref, hp_ref):
    h = jnp.dot(x_ref[...], w_ref[...], preferred_element_type=jnp.float32)
    h_ref[...] = h
    hp_ref[...] = _dis_from_deg(deg_ref)[:, None] * h


def _tc_mm1(x, w, deg2):
    n, d = x.shape
    b = 512
    return pl.pallas_call(
        _mm1_body,
        grid=(pl.cdiv(n, b),),
        in_specs=[pl.BlockSpec((b, d), lambda i: (i, 0)),
                  pl.BlockSpec((d, w.shape[1]), lambda i: (0, 0)),
                  pl.BlockSpec((NC, b, 128), lambda i: (0, i, 0))],
        out_specs=[pl.BlockSpec((b, d), lambda i: (i, 0)),
                   pl.BlockSpec((b, d), lambda i: (i, 0))],
        out_shape=[jax.ShapeDtypeStruct((n, d), jnp.float32),
                   jax.ShapeDtypeStruct((n, d), jnp.float32)],
    )(x, w, deg2)


def _mid_body(acc_ref, h_ref, deg_ref, b_ref, w_ref, h2_ref, hp2_ref):
    dis = _dis_from_deg(deg_ref)[:, None]
    s = acc_ref[0] + acc_ref[1]
    out1 = dis * s + (dis * dis) * h_ref[...] + b_ref[...]
    g = jnp.maximum(out1, 0.0)
    h2 = jnp.dot(g, w_ref[...], preferred_element_type=jnp.float32)
    h2_ref[...] = h2
    hp2_ref[...] = dis * h2


def _tc_mid(acc2, h1, deg2, b1, w2):
    n, d = h1.shape
    b = 512
    return pl.pallas_call(
        _mid_body,
        grid=(pl.cdiv(n, b),),
        in_specs=[pl.BlockSpec((NC, b, d), lambda i: (0, i, 0)),
                  pl.BlockSpec((b, d), lambda i: (i, 0)),
                  pl.BlockSpec((NC, b, 128), lambda i: (0, i, 0)),
                  pl.BlockSpec((1, d), lambda i: (0, 0)),
                  pl.BlockSpec((d, d), lambda i: (0, 0))],
        out_specs=[pl.BlockSpec((b, d), lambda i: (i, 0)),
                   pl.BlockSpec((b, d), lambda i: (i, 0))],
        out_shape=[jax.ShapeDtypeStruct((n, d), jnp.float32),
                   jax.ShapeDtypeStruct((n, d), jnp.float32)],
    )(acc2, h1, deg2, b1, w2)


def _final_body(acc_ref, h_ref, deg_ref, b_ref, o_ref):
    dis = _dis_from_deg(deg_ref)[:, None]
    s = acc_ref[0] + acc_ref[1]
    o_ref[...] = dis * s + (dis * dis) * h_ref[...] + b_ref[...]


def _tc_final(acc2, h2, deg2, b2):
    n, d = h2.shape
    b = 512
    return pl.pallas_call(
        _final_body,
        grid=(pl.cdiv(n, b),),
        in_specs=[pl.BlockSpec((NC, b, d), lambda i: (0, i, 0)),
                  pl.BlockSpec((b, d), lambda i: (i, 0)),
                  pl.BlockSpec((NC, b, 128), lambda i: (0, i, 0)),
                  pl.BlockSpec((1, d), lambda i: (0, 0))],
        out_specs=pl.BlockSpec((b, d), lambda i: (i, 0)),
        out_shape=jax.ShapeDtypeStruct((n, d), jnp.float32),
    )(acc2, h2, deg2, b2)


def kernel(x, edge_index, edge_weight, W1, b1, W2, b2):
    n = x.shape[0]
    e = edge_weight.shape[0]
    ep = _pad_up(e, NT * W)
    np_ = _pad_up(n, NS * W)
    pad = ep - e
    wpt = ep // (NT * W)

    src = jnp.concatenate([edge_index[0], jnp.zeros((pad,), jnp.int32)])
    dst = jnp.concatenate([edge_index[1], jnp.zeros((pad,), jnp.int32)])
    ew = jnp.concatenate([edge_weight, jnp.zeros((pad,), jnp.float32)])
    b1r = b1.reshape(1, -1)
    b2r = b2.reshape(1, -1)

    deg2 = _sc_deg(dst, ew, np_)
    h1, hp1 = _tc_mm1(x, W1, deg2)
    acc1 = _sc_msg(hp1, src, dst, ew, np_)[:, :n]
    h2, hp2 = _tc_mid(acc1, h1, deg2, b1r, W2)
    acc2 = _sc_msg(hp2, src, dst, ew, np_)[:, :n]
    return _tc_final(acc2, h2, deg2, b2r)
